# baseline (speedup 1.0000x reference)
import functools

import jax
import jax.numpy as jnp
from jax import lax
from jax.experimental import pallas as pl
from jax.experimental.pallas import tpu as pltpu
from jax.experimental.pallas import tpu_sc as plsc

_NUM_WORKERS = 32
_CHUNK = 128
_B = 16384 * 20
_D = 64
_PER_W = _B // _NUM_WORKERS
_NCHUNK = _PER_W // _CHUNK


@functools.partial(
    pl.kernel,
    out_type=jax.ShapeDtypeStruct((_B, _D), jnp.float32),
    mesh=plsc.VectorSubcoreMesh(core_axis_name="c", subcore_axis_name="s"),
    scratch_types=[
        pltpu.VMEM((_NCHUNK, _CHUNK), jnp.int32),
        pltpu.VMEM((_CHUNK, _D), jnp.float32),
        pltpu.SemaphoreType.DMA,
    ],
    compiler_params=pltpu.CompilerParams(use_tc_tiling_on_sc=False),
)
def _gather_kernel(idx_hbm, table_hbm, out_hbm, idx_v, rows_v, sem):
    wid = lax.axis_index("s") * 2 + lax.axis_index("c")
    base = wid * _PER_W
    pltpu.sync_copy(idx_hbm.at[wid], idx_v)

    @pl.loop(0, _NCHUNK)
    def _chunk(j):
        pltpu.async_copy(table_hbm.at[idx_v.at[j]], rows_v, sem).wait()
        pltpu.sync_copy(rows_v, out_hbm.at[pl.ds(base + j * _CHUNK, _CHUNK)])


def kernel(embed_id, weight):
    idx = embed_id.reshape(_NUM_WORKERS, _NCHUNK, _CHUNK)
    out = _gather_kernel(idx, weight)
    return out.reshape(16384, 20, _D)


# speedup vs baseline: 1.0679x; 1.0679x over previous
import functools

import jax
import jax.numpy as jnp
from jax import lax
from jax.experimental import pallas as pl
from jax.experimental.pallas import tpu as pltpu
from jax.experimental.pallas import tpu_sc as plsc

_NUM_WORKERS = 32
_CHUNK = 128
_B = 16384 * 20
_D = 64
_PER_W = _B // _NUM_WORKERS
_NCHUNK = _PER_W // _CHUNK
_NBUF = 8
_K = 4
_S = _NCHUNK // _NBUF


@functools.partial(
    pl.kernel,
    out_type=jax.ShapeDtypeStruct((_B, _D), jnp.float32),
    mesh=plsc.VectorSubcoreMesh(core_axis_name="c", subcore_axis_name="s"),
    scratch_types=[
        pltpu.VMEM((_NCHUNK, _CHUNK), jnp.int32),
        pltpu.VMEM((_NBUF, _CHUNK, _D), jnp.float32),
        pltpu.SemaphoreType.DMA((_NBUF,)),
        pltpu.SemaphoreType.DMA((_NBUF,)),
    ],
    compiler_params=pltpu.CompilerParams(use_tc_tiling_on_sc=False),
)
def _gather_kernel(idx_hbm, table_hbm, out_hbm, idx_v, bufs, gsem, osem):
    wid = lax.axis_index("s") * 2 + lax.axis_index("c")
    base = wid * _PER_W
    pltpu.sync_copy(idx_hbm.at[wid], idx_v)

    def out_slice(c):
        return out_hbm.at[pl.ds(base + c * _CHUNK, _CHUNK)]

    def fire_gather(g, b):
        pltpu.async_copy(table_hbm.at[idx_v.at[g]], bufs.at[b], gsem.at[b])

    @pl.loop(0, _S)
    def _super(s):
        for b in range(_NBUF):
            g = s * _NBUF + b

            @pl.when(s > 0)
            def _():
                pltpu.make_async_copy(bufs.at[b], out_slice(g - _NBUF),
                                      osem.at[b]).wait()

            fire_gather(g, b)

            bk = (b - _K) % _NBUF
            c = g - _K

            @pl.when(c >= 0)
            def _():
                pltpu.make_async_copy(table_hbm.at[idx_v.at[c]], bufs.at[bk],
                                      gsem.at[bk]).wait()
                pltpu.async_copy(bufs.at[bk], out_slice(c), osem.at[bk])

    for c in range(_NCHUNK - _K, _NCHUNK):
        b = c % _NBUF
        pltpu.make_async_copy(table_hbm.at[idx_v.at[c]], bufs.at[b],
                              gsem.at[b]).wait()
        pltpu.async_copy(bufs.at[b], out_slice(c), osem.at[b])
    for c in range(_NCHUNK - _NBUF, _NCHUNK):
        b = c % _NBUF
        pltpu.make_async_copy(bufs.at[b], out_slice(c), osem.at[b]).wait()


def kernel(embed_id, weight):
    idx = embed_id.reshape(_NUM_WORKERS, _NCHUNK, _CHUNK)
    out = _gather_kernel(idx, weight)
    return out.reshape(16384, 20, _D)
